# baseline (speedup 1.0000x reference)
import functools

import jax
import jax.numpy as jnp
from jax import lax
from jax.experimental import pallas as pl
from jax.experimental.pallas import tpu as pltpu
from jax.experimental.pallas import tpu_sc as plsc

B, S, D = 4096, 200, 128
NC, NS, L = 2, 16, 16
NW = NC * NS
BPW = B // NW

NCH = 4
CH = BPW // NCH

_mesh = plsc.VectorSubcoreMesh(core_axis_name="c", subcore_axis_name="s")


@functools.partial(
    pl.kernel,
    mesh=_mesh,
    out_type=jax.ShapeDtypeStruct((B, D), jnp.float32),
    scratch_types=[
        pltpu.VMEM((BPW,), jnp.int32),
        pltpu.VMEM((BPW, D), jnp.float32),
        pltpu.SemaphoreType.DMA,
        pltpu.SemaphoreType.DMA,
        pltpu.SemaphoreType.DMA,
        pltpu.SemaphoreType.DMA,
        pltpu.SemaphoreType.DMA,
    ],
)
def _gather_rows(table_hbm, idx_hbm, out_hbm, idx_v, rows_v,
                 g0, g1, g2, g3, wsem):
    wid = lax.axis_index("s") * NC + lax.axis_index("c")
    base = wid * BPW
    pltpu.sync_copy(idx_hbm.at[pl.ds(base, BPW)], idx_v)
    lane_off = lax.iota(jnp.int32, L) * S
    for i in range(BPW // L):
        sl = pl.ds(i * L, L)
        idx_v[sl] = idx_v[sl] + ((base + i * L) * S + lane_off)
    gsems = [g0, g1, g2, g3]
    gathers = []
    for c in range(NCH):
        gathers.append(pltpu.async_copy(
            table_hbm.at[idx_v.at[pl.ds(c * CH, CH)]],
            rows_v.at[pl.ds(c * CH, CH)], gsems[c]))
    writes = []
    for c in range(NCH):
        gathers[c].wait()
        writes.append(pltpu.async_copy(
            rows_v.at[pl.ds(c * CH, CH)],
            out_hbm.at[pl.ds(base + c * CH, CH)], wsem))
    for w in writes:
        w.wait()


def kernel(seq, obj_idx):
    table = seq.reshape(B * S, D)
    idx = obj_idx.astype(jnp.int32)
    return _gather_rows(table, idx)


# speedup vs baseline: 1.0144x; 1.0144x over previous
import functools

import jax
import jax.numpy as jnp
from jax import lax
from jax.experimental import pallas as pl
from jax.experimental.pallas import tpu as pltpu
from jax.experimental.pallas import tpu_sc as plsc

B, S, D = 4096, 200, 128
NC, NS, L = 2, 16, 16
NW = NC * NS
BPW = B // NW

NCH = 2
CH = BPW // NCH

_mesh = plsc.VectorSubcoreMesh(core_axis_name="c", subcore_axis_name="s")


@functools.partial(
    pl.kernel,
    mesh=_mesh,
    out_type=jax.ShapeDtypeStruct((B, D), jnp.float32),
    scratch_types=[
        pltpu.VMEM((BPW,), jnp.int32),
        pltpu.VMEM((BPW, D), jnp.float32),
        pltpu.SemaphoreType.DMA,
        pltpu.SemaphoreType.DMA,
        pltpu.SemaphoreType.DMA,
    ],
)
def _gather_rows(table_hbm, idx_hbm, out_hbm, idx_v, rows_v, g0, g1, wsem):
    wid = lax.axis_index("s") * NC + lax.axis_index("c")
    base = wid * BPW
    pltpu.sync_copy(idx_hbm.at[pl.ds(base, BPW)], idx_v)
    lane_off = lax.iota(jnp.int32, L) * S
    for i in range(BPW // L):
        sl = pl.ds(i * L, L)
        idx_v[sl] = idx_v[sl] + ((base + i * L) * S + lane_off)
    ga = pltpu.async_copy(table_hbm.at[idx_v.at[pl.ds(0, CH)]],
                          rows_v.at[pl.ds(0, CH)], g0)
    gb = pltpu.async_copy(table_hbm.at[idx_v.at[pl.ds(CH, CH)]],
                          rows_v.at[pl.ds(CH, CH)], g1)
    ga.wait()
    wa = pltpu.async_copy(rows_v.at[pl.ds(0, CH)],
                          out_hbm.at[pl.ds(base, CH)], wsem)
    gb.wait()
    wb = pltpu.async_copy(rows_v.at[pl.ds(CH, CH)],
                          out_hbm.at[pl.ds(base + CH, CH)], wsem)
    wa.wait()
    wb.wait()


def kernel(seq, obj_idx):
    table = seq.reshape(B * S, D)
    idx = obj_idx.astype(jnp.int32)
    return _gather_rows(table, idx)


# speedup vs baseline: 1.0184x; 1.0039x over previous
import functools

import jax
import jax.numpy as jnp
from jax import lax
from jax.experimental import pallas as pl
from jax.experimental.pallas import tpu as pltpu
from jax.experimental.pallas import tpu_sc as plsc

B, S, D = 4096, 200, 128
NC, NS, L = 2, 16, 16
NW = NC * NS
BPW = B // NW

_mesh = plsc.VectorSubcoreMesh(core_axis_name="c", subcore_axis_name="s")


@functools.partial(
    pl.kernel,
    mesh=_mesh,
    out_type=jax.ShapeDtypeStruct((B, D), jnp.float32),
    scratch_types=[
        pltpu.VMEM((BPW,), jnp.int32),
        pltpu.VMEM((BPW, D), jnp.float32),
        pltpu.SemaphoreType.DMA,
    ],
)
def _gather_rows(table_hbm, idx_hbm, out_hbm, idx_v, rows_v, sem):
    wid = lax.axis_index("s") * NC + lax.axis_index("c")
    base = wid * BPW
    pltpu.sync_copy(idx_hbm.at[pl.ds(base, BPW)], idx_v)
    lane_off = lax.iota(jnp.int32, L) * S
    for i in range(BPW // L):
        sl = pl.ds(i * L, L)
        idx_v[sl] = idx_v[sl] + ((base + i * L) * S + lane_off)
    pltpu.async_copy(table_hbm.at[idx_v], rows_v, sem).wait()
    pltpu.sync_copy(rows_v, out_hbm.at[pl.ds(base, BPW)])


def kernel(seq, obj_idx):
    table = seq.reshape(B * S, D)
    idx = obj_idx.astype(jnp.int32)
    return _gather_rows(table, idx)
